# baseline (speedup 1.0000x reference)
import functools

import jax
import jax.numpy as jnp
from jax import lax
from jax.experimental import pallas as pl
from jax.experimental.pallas import tpu as pltpu
from jax.experimental.pallas import tpu_sc as plsc

N_ROWS = 32768
D = 2048
K = 512
L = 16
VECS = D // L
NC = 2
NS = 16
NW = NC * NS
N_SC = 256
ROWS_PER_W = N_SC // NW
CHUNK = 8


def _radix_pass(xin, hist, base, prefix, r, s):
    ones = jnp.ones((L,), jnp.int32)
    zeros = jnp.zeros((L,), jnp.int32)

    @plsc.parallel_loop(0, 16, unroll=4)
    def _clear(h):
        hist[pl.ds(h * L, L)] = zeros

    pfx_hi = lax.shift_right_logical(prefix, s + 8) if s != 24 else None

    @plsc.parallel_loop(0, VECS, unroll=8)
    def _hist_pass(v):
        x = xin[pl.ds(base + v * L, L)]
        a = lax.bitcast_convert_type(x, jnp.int32) & jnp.int32(0x7FFFFFFF)
        rev = (lax.shift_right_logical(a, s) & jnp.int32(0xFF)) ^ jnp.int32(0xFF)
        if s == 24:
            plsc.addupdate_scatter(hist, [rev], ones)
        else:
            m = lax.shift_right_logical(a, s + 8) == pfx_hi
            plsc.addupdate_scatter(hist, [rev], ones, mask=m)

    cums = []
    for h in range(16):
        hv = hist[pl.ds(h * L, L)]
        cums.append(plsc.cumsum(hv))
    run = jnp.int32(0)
    acc_rev = zeros
    acc_hi = zeros
    acc_lo = jnp.full((L,), jnp.int32(0x7FFFFFFF))
    for h in range(16):
        c = cums[h] + run
        run = run + cums[h][15]
        lt = c < r
        acc_rev = acc_rev + lt.astype(jnp.int32)
        acc_hi = jnp.maximum(acc_hi, jnp.where(lt, c, jnp.int32(0)))
        acc_lo = jnp.minimum(acc_lo, jnp.where(lt, jnp.int32(0x7FFFFFFF), c))
    revstar = jnp.sum(acc_rev)
    hi = jnp.max(acc_hi)
    cnt = jnp.min(acc_lo) - hi
    r = r - hi
    prefix = prefix | lax.shift_left(jnp.int32(255) - revstar, s)
    return prefix, r, cnt


def _select_threshold(xin, hist, cbuf, tref, base):
    prefix = jnp.int32(0)
    r = jnp.int32(K)
    for s in (24, 16):
        prefix, r, cnt = _radix_pass(xin, hist, base, prefix, r, s)

    pfx_hi16 = lax.shift_right_logical(prefix, 16)
    lanes = lax.iota(jnp.int32, L)

    @pl.when(cnt <= L)
    def _small():
        @plsc.parallel_loop(0, VECS, unroll=8, carry=jnp.int32(0))
        def _compact(v, off):
            x = xin[pl.ds(base + v * L, L)]
            a = lax.bitcast_convert_type(x, jnp.int32) & jnp.int32(0x7FFFFFFF)
            m = lax.shift_right_logical(a, 16) == pfx_hi16
            plsc.store_compressed(cbuf.at[pl.ds(off, L)], a, mask=m)
            return off + plsc.all_reduce_population_count(m)[0]

        vals = jnp.where(lanes < cnt, cbuf[pl.ds(0, L)], jnp.int32(0))
        skey, _ = plsc.sort_key_val(vals, vals, descending=True)
        tref[0] = jnp.max(jnp.where(lanes == r - 1, skey, jnp.int32(0)))

    @pl.when(cnt > L)
    def _big():
        p2 = prefix
        r2 = r
        for s in (8, 0):
            p2, r2, _ = _radix_pass(xin, hist, base, p2, r2, s)
        tref[0] = p2

    return tref[0]


NCHUNKS = ROWS_PER_W // CHUNK


def _tec_body(x_hbm, ml_hbm, out_hbm, xin0, xin1, xout0, xout1, maskv, hist,
              cbuf, tref, isem0, isem1, osem0, osem1):
    wid = lax.axis_index("s") * NC + lax.axis_index("c")
    row0 = wid * ROWS_PER_W
    xins = (xin0, xin1)
    xouts = (xout0, xout1)
    isems = (isem0, isem1)
    osems = (osem0, osem1)

    pltpu.sync_copy(ml_hbm, maskv)
    for v in range(VECS):
        z = maskv[pl.ds(v * L, L)]
        maskv[pl.ds(v * L, L)] = 1.0 / (1.0 + jnp.exp(-z))

    def _in_slice(c):
        return x_hbm.at[pl.ds((row0 + c * CHUNK) * D, CHUNK * D)]

    for b in range(2):
        pltpu.async_copy(_in_slice(b), xins[b], isems[b])

    @pl.loop(0, NCHUNKS, step=2)
    def _chunk(c0):
        for b in range(2):
            c = c0 + b
            xin, xout = xins[b], xouts[b]
            pltpu.make_async_copy(_in_slice(c), xin, isems[b]).wait()

            @pl.when(c >= 2)
            def _():
                pltpu.make_async_copy(
                    xout, out_hbm.at[pl.ds((row0 + (c - 2) * CHUNK) * D, CHUNK * D)],
                    osems[b]).wait()

            @pl.loop(0, CHUNK)
            def _row(ri):
                base = ri * D
                thresh = _select_threshold(xin, hist, cbuf, tref, base)

                @plsc.parallel_loop(0, VECS, unroll=8)
                def _write(v):
                    x = xin[pl.ds(base + v * L, L)]
                    a = lax.bitcast_convert_type(x, jnp.int32) & jnp.int32(0x7FFFFFFF)
                    mv = maskv[pl.ds(v * L, L)]
                    xout[pl.ds(base + v * L, L)] = jnp.where(a >= thresh, x * mv, 0.0)

            pltpu.async_copy(
                xout, out_hbm.at[pl.ds((row0 + c * CHUNK) * D, CHUNK * D)], osems[b])

            @pl.when(c + 2 < NCHUNKS)
            def _():
                pltpu.async_copy(_in_slice(c + 2), xin, isems[b])

    for b in range(2):
        pltpu.make_async_copy(
            xouts[b],
            out_hbm.at[pl.ds((row0 + (NCHUNKS - 2 + b) * CHUNK) * D, CHUNK * D)],
            osems[b]).wait()


_sc_topk = functools.partial(
    pl.kernel,
    out_type=jax.ShapeDtypeStruct((N_SC * D,), jnp.float32),
    mesh=plsc.VectorSubcoreMesh(core_axis_name="c", subcore_axis_name="s"),
    compiler_params=pltpu.CompilerParams(needs_layout_passes=False),
    scratch_types=[
        pltpu.VMEM((CHUNK * D,), jnp.float32),
        pltpu.VMEM((CHUNK * D,), jnp.float32),
        pltpu.VMEM((CHUNK * D,), jnp.float32),
        pltpu.VMEM((CHUNK * D,), jnp.float32),
        pltpu.VMEM((D,), jnp.float32),
        pltpu.VMEM((256,), jnp.int32),
        pltpu.VMEM((2 * L,), jnp.int32),
        pltpu.SMEM((8,), jnp.int32),
        pltpu.SemaphoreType.DMA,
        pltpu.SemaphoreType.DMA,
        pltpu.SemaphoreType.DMA,
        pltpu.SemaphoreType.DMA,
    ],
)(_tec_body)


BR = 256
N_TC = N_ROWS - N_SC


def _tc_block(x_ref, ml_ref, o_ref):
    x = x_ref[...]
    a = lax.bitcast_convert_type(x, jnp.int32) & jnp.int32(0x7FFFFFFF)
    lo = jnp.zeros((BR, 1), jnp.int32)
    hi = jnp.full((BR, 1), jnp.int32(0x7F800001))

    def body(_, lohi):
        lo, hi = lohi
        mid = lax.shift_right_logical(lo + hi, 1)
        cnt = jnp.sum((a >= mid).astype(jnp.int32), axis=1, keepdims=True)
        pred = cnt >= K
        return jnp.where(pred, mid, lo), jnp.where(pred, hi, mid)

    lo, hi = lax.fori_loop(0, 31, body, (lo, hi))
    mask = jax.nn.sigmoid(ml_ref[...])
    o_ref[...] = jnp.where(a >= lo, x * mask, 0.0)


_tc_topk = pl.pallas_call(
    _tc_block,
    grid=(N_TC // BR,),
    in_specs=[
        pl.BlockSpec((BR, D), lambda i: (N_SC // BR + i, 0)),
        pl.BlockSpec((1, D), lambda i: (0, 0)),
    ],
    out_specs=pl.BlockSpec((BR, D), lambda i: (i, 0)),
    out_shape=jax.ShapeDtypeStruct((N_TC, D), jnp.float32),
)


@jax.jit
def kernel(x, mask_logits):
    sc_out = _sc_topk(x.reshape(-1), mask_logits)
    tc_out = _tc_topk(x, mask_logits.reshape(1, D))
    return jnp.concatenate([sc_out.reshape(N_SC, D), tc_out], axis=0)


# speedup vs baseline: 1.4365x; 1.4365x over previous
import functools

import jax
import jax.numpy as jnp
from jax import lax
from jax.experimental import pallas as pl
from jax.experimental.pallas import tpu as pltpu
from jax.experimental.pallas import tpu_sc as plsc

N_ROWS = 32768
D = 2048
K = 512
L = 16
VECS = D // L
NC = 2
NS = 16
NW = NC * NS
N_SC = 19456
ROWS_PER_W = N_SC // NW
CHUNK = 8
assert ROWS_PER_W % (2 * CHUNK) == 0 and N_SC % NW == 0


def _radix_pass(xin, hist, base, prefix, r, s):
    ones = jnp.ones((L,), jnp.int32)
    zeros = jnp.zeros((L,), jnp.int32)

    @plsc.parallel_loop(0, 16, unroll=4)
    def _clear(h):
        hist[pl.ds(h * L, L)] = zeros

    pfx_hi = lax.shift_right_logical(prefix, s + 8) if s != 24 else None

    @plsc.parallel_loop(0, VECS, unroll=8)
    def _hist_pass(v):
        x = xin[pl.ds(base + v * L, L)]
        a = lax.bitcast_convert_type(x, jnp.int32) & jnp.int32(0x7FFFFFFF)
        rev = (lax.shift_right_logical(a, s) & jnp.int32(0xFF)) ^ jnp.int32(0xFF)
        if s == 24:
            plsc.addupdate_scatter(hist, [rev], ones)
        else:
            m = lax.shift_right_logical(a, s + 8) == pfx_hi
            plsc.addupdate_scatter(hist, [rev], ones, mask=m)

    cums = []
    for h in range(16):
        hv = hist[pl.ds(h * L, L)]
        cums.append(plsc.cumsum(hv))
    run = jnp.int32(0)
    acc_rev = zeros
    acc_hi = zeros
    acc_lo = jnp.full((L,), jnp.int32(0x7FFFFFFF))
    for h in range(16):
        c = cums[h] + run
        run = run + cums[h][15]
        lt = c < r
        acc_rev = acc_rev + lt.astype(jnp.int32)
        acc_hi = jnp.maximum(acc_hi, jnp.where(lt, c, jnp.int32(0)))
        acc_lo = jnp.minimum(acc_lo, jnp.where(lt, jnp.int32(0x7FFFFFFF), c))
    revstar = jnp.sum(acc_rev)
    hi = jnp.max(acc_hi)
    cnt = jnp.min(acc_lo) - hi
    r = r - hi
    prefix = prefix | lax.shift_left(jnp.int32(255) - revstar, s)
    return prefix, r, cnt


def _select_threshold(xin, hist, cbuf, tref, base):
    prefix = jnp.int32(0)
    r = jnp.int32(K)
    for s in (24, 16):
        prefix, r, cnt = _radix_pass(xin, hist, base, prefix, r, s)

    pfx_hi16 = lax.shift_right_logical(prefix, 16)
    lanes = lax.iota(jnp.int32, L)

    @pl.when(cnt <= L)
    def _small():
        @plsc.parallel_loop(0, VECS, unroll=8, carry=jnp.int32(0))
        def _compact(v, off):
            x = xin[pl.ds(base + v * L, L)]
            a = lax.bitcast_convert_type(x, jnp.int32) & jnp.int32(0x7FFFFFFF)
            m = lax.shift_right_logical(a, 16) == pfx_hi16
            plsc.store_compressed(cbuf.at[pl.ds(off, L)], a, mask=m)
            return off + plsc.all_reduce_population_count(m)[0]

        vals = jnp.where(lanes < cnt, cbuf[pl.ds(0, L)], jnp.int32(0))
        skey, _ = plsc.sort_key_val(vals, vals, descending=True)
        tref[0] = jnp.max(jnp.where(lanes == r - 1, skey, jnp.int32(0)))

    @pl.when(cnt > L)
    def _big():
        p2 = prefix
        r2 = r
        for s in (8, 0):
            p2, r2, _ = _radix_pass(xin, hist, base, p2, r2, s)
        tref[0] = p2

    return tref[0]


NCHUNKS = ROWS_PER_W // CHUNK


def _tec_body(x_hbm, ml_hbm, out_hbm, xin0, xin1, xout0, xout1, maskv, hist,
              cbuf, tref, isem0, isem1, osem0, osem1):
    wid = lax.axis_index("s") * NC + lax.axis_index("c")
    row0 = wid * ROWS_PER_W
    xins = (xin0, xin1)
    xouts = (xout0, xout1)
    isems = (isem0, isem1)
    osems = (osem0, osem1)

    pltpu.sync_copy(ml_hbm, maskv)
    for v in range(VECS):
        z = maskv[pl.ds(v * L, L)]
        maskv[pl.ds(v * L, L)] = 1.0 / (1.0 + jnp.exp(-z))

    def _in_slice(c):
        return x_hbm.at[pl.ds((row0 + c * CHUNK) * D, CHUNK * D)]

    for b in range(2):
        pltpu.async_copy(_in_slice(b), xins[b], isems[b])

    @pl.loop(0, NCHUNKS, step=2)
    def _chunk(c0):
        for b in range(2):
            c = c0 + b
            xin, xout = xins[b], xouts[b]
            pltpu.make_async_copy(_in_slice(c), xin, isems[b]).wait()

            @pl.when(c >= 2)
            def _():
                pltpu.make_async_copy(
                    xout, out_hbm.at[pl.ds((row0 + (c - 2) * CHUNK) * D, CHUNK * D)],
                    osems[b]).wait()

            @pl.loop(0, CHUNK)
            def _row(ri):
                base = ri * D
                thresh = _select_threshold(xin, hist, cbuf, tref, base)

                @plsc.parallel_loop(0, VECS, unroll=8)
                def _write(v):
                    x = xin[pl.ds(base + v * L, L)]
                    a = lax.bitcast_convert_type(x, jnp.int32) & jnp.int32(0x7FFFFFFF)
                    mv = maskv[pl.ds(v * L, L)]
                    xout[pl.ds(base + v * L, L)] = jnp.where(a >= thresh, x * mv, 0.0)

            pltpu.async_copy(
                xout, out_hbm.at[pl.ds((row0 + c * CHUNK) * D, CHUNK * D)], osems[b])

            @pl.when(c + 2 < NCHUNKS)
            def _():
                pltpu.async_copy(_in_slice(c + 2), xin, isems[b])

    for b in range(2):
        pltpu.make_async_copy(
            xouts[b],
            out_hbm.at[pl.ds((row0 + (NCHUNKS - 2 + b) * CHUNK) * D, CHUNK * D)],
            osems[b]).wait()


_sc_topk = functools.partial(
    pl.kernel,
    out_type=jax.ShapeDtypeStruct((N_SC * D,), jnp.float32),
    mesh=plsc.VectorSubcoreMesh(core_axis_name="c", subcore_axis_name="s"),
    compiler_params=pltpu.CompilerParams(needs_layout_passes=False),
    scratch_types=[
        pltpu.VMEM((CHUNK * D,), jnp.float32),
        pltpu.VMEM((CHUNK * D,), jnp.float32),
        pltpu.VMEM((CHUNK * D,), jnp.float32),
        pltpu.VMEM((CHUNK * D,), jnp.float32),
        pltpu.VMEM((D,), jnp.float32),
        pltpu.VMEM((256,), jnp.int32),
        pltpu.VMEM((2 * L,), jnp.int32),
        pltpu.SMEM((8,), jnp.int32),
        pltpu.SemaphoreType.DMA,
        pltpu.SemaphoreType.DMA,
        pltpu.SemaphoreType.DMA,
        pltpu.SemaphoreType.DMA,
    ],
)(_tec_body)


BR = 256
N_TC = N_ROWS - N_SC


def _tc_block(x_ref, ml_ref, o_ref):
    x = x_ref[...]
    a = lax.bitcast_convert_type(x, jnp.int32) & jnp.int32(0x7FFFFFFF)
    lo = jnp.zeros((BR, 1), jnp.int32)
    hi = jnp.full((BR, 1), jnp.int32(0x7F800001))

    def body(_, lohi):
        lo, hi = lohi
        mid = lax.shift_right_logical(lo + hi, 1)
        cnt = jnp.sum((a >= mid).astype(jnp.int32), axis=1, keepdims=True)
        pred = cnt >= K
        return jnp.where(pred, mid, lo), jnp.where(pred, hi, mid)

    lo, hi = lax.fori_loop(0, 31, body, (lo, hi))
    mask = jax.nn.sigmoid(ml_ref[...])
    o_ref[...] = jnp.where(a >= lo, x * mask, 0.0)


_tc_topk = pl.pallas_call(
    _tc_block,
    grid=(N_TC // BR,),
    in_specs=[
        pl.BlockSpec((BR, D), lambda i: (N_SC // BR + i, 0)),
        pl.BlockSpec((1, D), lambda i: (0, 0)),
    ],
    out_specs=pl.BlockSpec((BR, D), lambda i: (i, 0)),
    out_shape=jax.ShapeDtypeStruct((N_TC, D), jnp.float32),
)


@jax.jit
def kernel(x, mask_logits):
    sc_out = _sc_topk(x.reshape(-1), mask_logits)
    tc_out = _tc_topk(x, mask_logits.reshape(1, D))
    return jnp.concatenate([sc_out.reshape(N_SC, D), tc_out], axis=0)


# speedup vs baseline: 1.4678x; 1.0218x over previous
import functools

import jax
import jax.numpy as jnp
from jax import lax
from jax.experimental import pallas as pl
from jax.experimental.pallas import tpu as pltpu
from jax.experimental.pallas import tpu_sc as plsc

N_ROWS = 32768
D = 2048
K = 512
L = 16
VECS = D // L
NC = 2
NS = 16
NW = NC * NS
N_SC = 15360
ROWS_PER_W = N_SC // NW
CHUNK = 8
assert ROWS_PER_W % (2 * CHUNK) == 0 and N_SC % NW == 0


def _radix_pass(xin, hist, base, prefix, r, s):
    ones = jnp.ones((L,), jnp.int32)
    zeros = jnp.zeros((L,), jnp.int32)

    @plsc.parallel_loop(0, 16, unroll=4)
    def _clear(h):
        hist[pl.ds(h * L, L)] = zeros

    pfx_hi = lax.shift_right_logical(prefix, s + 8) if s != 24 else None

    @plsc.parallel_loop(0, VECS, unroll=8)
    def _hist_pass(v):
        x = xin[pl.ds(base + v * L, L)]
        a = lax.bitcast_convert_type(x, jnp.int32) & jnp.int32(0x7FFFFFFF)
        rev = (lax.shift_right_logical(a, s) & jnp.int32(0xFF)) ^ jnp.int32(0xFF)
        if s == 24:
            plsc.addupdate_scatter(hist, [rev], ones)
        else:
            m = lax.shift_right_logical(a, s + 8) == pfx_hi
            plsc.addupdate_scatter(hist, [rev], ones, mask=m)

    cums = []
    for h in range(16):
        hv = hist[pl.ds(h * L, L)]
        cums.append(plsc.cumsum(hv))
    run = jnp.int32(0)
    acc_rev = zeros
    acc_hi = zeros
    acc_lo = jnp.full((L,), jnp.int32(0x7FFFFFFF))
    for h in range(16):
        c = cums[h] + run
        run = run + cums[h][15]
        lt = c < r
        acc_rev = acc_rev + lt.astype(jnp.int32)
        acc_hi = jnp.maximum(acc_hi, jnp.where(lt, c, jnp.int32(0)))
        acc_lo = jnp.minimum(acc_lo, jnp.where(lt, jnp.int32(0x7FFFFFFF), c))
    revstar = jnp.sum(acc_rev)
    hi = jnp.max(acc_hi)
    cnt = jnp.min(acc_lo) - hi
    r = r - hi
    prefix = prefix | lax.shift_left(jnp.int32(255) - revstar, s)
    return prefix, r, cnt


def _select_threshold(xin, hist, cbuf, tref, base):
    prefix = jnp.int32(0)
    r = jnp.int32(K)
    for s in (24, 16):
        prefix, r, cnt = _radix_pass(xin, hist, base, prefix, r, s)

    pfx_hi16 = lax.shift_right_logical(prefix, 16)
    lanes = lax.iota(jnp.int32, L)

    @pl.when(cnt <= L)
    def _small():
        @plsc.parallel_loop(0, VECS, unroll=8, carry=jnp.int32(0))
        def _compact(v, off):
            x = xin[pl.ds(base + v * L, L)]
            a = lax.bitcast_convert_type(x, jnp.int32) & jnp.int32(0x7FFFFFFF)
            m = lax.shift_right_logical(a, 16) == pfx_hi16
            plsc.store_compressed(cbuf.at[pl.ds(off, L)], a, mask=m)
            return off + plsc.all_reduce_population_count(m)[0]

        vals = jnp.where(lanes < cnt, cbuf[pl.ds(0, L)], jnp.int32(0))
        skey, _ = plsc.sort_key_val(vals, vals, descending=True)
        tref[0] = jnp.max(jnp.where(lanes == r - 1, skey, jnp.int32(0)))

    @pl.when(cnt > L)
    def _big():
        p2 = prefix
        r2 = r
        for s in (8, 0):
            p2, r2, _ = _radix_pass(xin, hist, base, p2, r2, s)
        tref[0] = p2

    return tref[0]


NCHUNKS = ROWS_PER_W // CHUNK


def _tec_body(x_hbm, ml_hbm, out_hbm, xin0, xin1, xout0, xout1, maskv, hist,
              cbuf, tref, isem0, isem1, osem0, osem1):
    wid = lax.axis_index("s") * NC + lax.axis_index("c")
    row0 = wid * ROWS_PER_W
    xins = (xin0, xin1)
    xouts = (xout0, xout1)
    isems = (isem0, isem1)
    osems = (osem0, osem1)

    pltpu.sync_copy(ml_hbm, maskv)
    for v in range(VECS):
        z = maskv[pl.ds(v * L, L)]
        maskv[pl.ds(v * L, L)] = 1.0 / (1.0 + jnp.exp(-z))

    def _in_slice(c):
        return x_hbm.at[pl.ds((row0 + c * CHUNK) * D, CHUNK * D)]

    for b in range(2):
        pltpu.async_copy(_in_slice(b), xins[b], isems[b])

    @pl.loop(0, NCHUNKS, step=2)
    def _chunk(c0):
        for b in range(2):
            c = c0 + b
            xin, xout = xins[b], xouts[b]
            pltpu.make_async_copy(_in_slice(c), xin, isems[b]).wait()

            @pl.when(c >= 2)
            def _():
                pltpu.make_async_copy(
                    xout, out_hbm.at[pl.ds((row0 + (c - 2) * CHUNK) * D, CHUNK * D)],
                    osems[b]).wait()

            @pl.loop(0, CHUNK)
            def _row(ri):
                base = ri * D
                thresh = _select_threshold(xin, hist, cbuf, tref, base)

                @plsc.parallel_loop(0, VECS, unroll=8)
                def _write(v):
                    x = xin[pl.ds(base + v * L, L)]
                    a = lax.bitcast_convert_type(x, jnp.int32) & jnp.int32(0x7FFFFFFF)
                    mv = maskv[pl.ds(v * L, L)]
                    xout[pl.ds(base + v * L, L)] = jnp.where(a >= thresh, x * mv, 0.0)

            pltpu.async_copy(
                xout, out_hbm.at[pl.ds((row0 + c * CHUNK) * D, CHUNK * D)], osems[b])

            @pl.when(c + 2 < NCHUNKS)
            def _():
                pltpu.async_copy(_in_slice(c + 2), xin, isems[b])

    for b in range(2):
        pltpu.make_async_copy(
            xouts[b],
            out_hbm.at[pl.ds((row0 + (NCHUNKS - 2 + b) * CHUNK) * D, CHUNK * D)],
            osems[b]).wait()


_sc_topk = functools.partial(
    pl.kernel,
    out_type=jax.ShapeDtypeStruct((N_SC * D,), jnp.float32),
    mesh=plsc.VectorSubcoreMesh(core_axis_name="c", subcore_axis_name="s"),
    compiler_params=pltpu.CompilerParams(needs_layout_passes=False),
    scratch_types=[
        pltpu.VMEM((CHUNK * D,), jnp.float32),
        pltpu.VMEM((CHUNK * D,), jnp.float32),
        pltpu.VMEM((CHUNK * D,), jnp.float32),
        pltpu.VMEM((CHUNK * D,), jnp.float32),
        pltpu.VMEM((D,), jnp.float32),
        pltpu.VMEM((256,), jnp.int32),
        pltpu.VMEM((2 * L,), jnp.int32),
        pltpu.SMEM((8,), jnp.int32),
        pltpu.SemaphoreType.DMA,
        pltpu.SemaphoreType.DMA,
        pltpu.SemaphoreType.DMA,
        pltpu.SemaphoreType.DMA,
    ],
)(_tec_body)


BR = 256
N_TC = N_ROWS - N_SC


def _tc_block(x_ref, ml_ref, o_ref):
    x = x_ref[...]
    a = lax.bitcast_convert_type(x, jnp.int32) & jnp.int32(0x7FFFFFFF)
    lo = jnp.zeros((BR, 1), jnp.int32)
    hi = jnp.full((BR, 1), jnp.int32(0x7F800001))

    def body(_, lohi):
        lo, hi = lohi
        mid = lax.shift_right_logical(lo + hi, 1)
        cnt = jnp.sum((a >= mid).astype(jnp.int32), axis=1, keepdims=True)
        pred = cnt >= K
        return jnp.where(pred, mid, lo), jnp.where(pred, hi, mid)

    lo, hi = lax.fori_loop(0, 31, body, (lo, hi))
    mask = jax.nn.sigmoid(ml_ref[...])
    o_ref[...] = jnp.where(a >= lo, x * mask, 0.0)


_tc_topk = pl.pallas_call(
    _tc_block,
    grid=(N_TC // BR,),
    in_specs=[
        pl.BlockSpec((BR, D), lambda i: (N_SC // BR + i, 0)),
        pl.BlockSpec((1, D), lambda i: (0, 0)),
    ],
    out_specs=pl.BlockSpec((BR, D), lambda i: (i, 0)),
    out_shape=jax.ShapeDtypeStruct((N_TC, D), jnp.float32),
)


@jax.jit
def kernel(x, mask_logits):
    sc_out = _sc_topk(x.reshape(-1), mask_logits)
    tc_out = _tc_topk(x, mask_logits.reshape(1, D))
    return jnp.concatenate([sc_out.reshape(N_SC, D), tc_out], axis=0)


# speedup vs baseline: 1.5170x; 1.0336x over previous
import functools

import jax
import jax.numpy as jnp
from jax import lax
from jax.experimental import pallas as pl
from jax.experimental.pallas import tpu as pltpu
from jax.experimental.pallas import tpu_sc as plsc

N_ROWS = 32768
D = 2048
K = 512
L = 16
VECS = D // L
NC = 2
NS = 16
NW = NC * NS
N_SC = 16384
ROWS_PER_W = N_SC // NW
CHUNK = 8
assert ROWS_PER_W % (2 * CHUNK) == 0 and N_SC % NW == 0


def _radix_pass(xin, hist, base, prefix, r, s):
    ones = jnp.ones((L,), jnp.int32)
    zeros = jnp.zeros((L,), jnp.int32)

    @plsc.parallel_loop(0, 16, unroll=4)
    def _clear(h):
        hist[pl.ds(h * L, L)] = zeros

    pfx_hi = lax.shift_right_logical(prefix, s + 8) if s != 24 else None

    @plsc.parallel_loop(0, VECS, unroll=8)
    def _hist_pass(v):
        x = xin[pl.ds(base + v * L, L)]
        a = lax.bitcast_convert_type(x, jnp.int32) & jnp.int32(0x7FFFFFFF)
        rev = (lax.shift_right_logical(a, s) & jnp.int32(0xFF)) ^ jnp.int32(0xFF)
        if s == 24:
            plsc.addupdate_scatter(hist, [rev], ones)
        else:
            m = lax.shift_right_logical(a, s + 8) == pfx_hi
            plsc.addupdate_scatter(hist, [rev], ones, mask=m)

    cums = []
    for h in range(16):
        hv = hist[pl.ds(h * L, L)]
        cums.append(plsc.cumsum(hv))
    run = jnp.int32(0)
    acc_rev = zeros
    acc_hi = zeros
    acc_lo = jnp.full((L,), jnp.int32(0x7FFFFFFF))
    for h in range(16):
        c = cums[h] + run
        run = run + cums[h][15]
        lt = c < r
        acc_rev = acc_rev + lt.astype(jnp.int32)
        acc_hi = jnp.maximum(acc_hi, jnp.where(lt, c, jnp.int32(0)))
        acc_lo = jnp.minimum(acc_lo, jnp.where(lt, jnp.int32(0x7FFFFFFF), c))
    revstar = jnp.sum(acc_rev)
    hi = jnp.max(acc_hi)
    cnt = jnp.min(acc_lo) - hi
    r = r - hi
    prefix = prefix | lax.shift_left(jnp.int32(255) - revstar, s)
    return prefix, r, cnt


def _select_threshold(xin, hist, cbuf, tref, base):
    prefix = jnp.int32(0)
    r = jnp.int32(K)
    for s in (24, 16):
        prefix, r, cnt = _radix_pass(xin, hist, base, prefix, r, s)

    pfx_hi16 = lax.shift_right_logical(prefix, 16)
    lanes = lax.iota(jnp.int32, L)

    @pl.when(cnt <= L)
    def _small():
        @plsc.parallel_loop(0, VECS, unroll=8, carry=jnp.int32(0))
        def _compact(v, off):
            x = xin[pl.ds(base + v * L, L)]
            a = lax.bitcast_convert_type(x, jnp.int32) & jnp.int32(0x7FFFFFFF)
            m = lax.shift_right_logical(a, 16) == pfx_hi16
            plsc.store_compressed(cbuf.at[pl.ds(off, L)], a, mask=m)
            return off + plsc.all_reduce_population_count(m)[0]

        vals = jnp.where(lanes < cnt, cbuf[pl.ds(0, L)], jnp.int32(0))
        skey, _ = plsc.sort_key_val(vals, vals, descending=True)
        tref[0] = jnp.max(jnp.where(lanes == r - 1, skey, jnp.int32(0)))

    @pl.when(cnt > L)
    def _big():
        p2 = prefix
        r2 = r
        for s in (8, 0):
            p2, r2, _ = _radix_pass(xin, hist, base, p2, r2, s)
        tref[0] = p2

    return tref[0]


NCHUNKS = ROWS_PER_W // CHUNK


def _tec_body(x_hbm, ml_hbm, out_hbm, xin0, xin1, xout0, xout1, maskv, hist,
              cbuf, tref, isem0, isem1, osem0, osem1):
    wid = lax.axis_index("s") * NC + lax.axis_index("c")
    row0 = wid * ROWS_PER_W
    xins = (xin0, xin1)
    xouts = (xout0, xout1)
    isems = (isem0, isem1)
    osems = (osem0, osem1)

    pltpu.sync_copy(ml_hbm, maskv)
    for v in range(VECS):
        z = maskv[pl.ds(v * L, L)]
        maskv[pl.ds(v * L, L)] = 1.0 / (1.0 + jnp.exp(-z))

    def _in_slice(c):
        return x_hbm.at[pl.ds((row0 + c * CHUNK) * D, CHUNK * D)]

    for b in range(2):
        pltpu.async_copy(_in_slice(b), xins[b], isems[b])

    @pl.loop(0, NCHUNKS, step=2)
    def _chunk(c0):
        for b in range(2):
            c = c0 + b
            xin, xout = xins[b], xouts[b]
            pltpu.make_async_copy(_in_slice(c), xin, isems[b]).wait()

            @pl.when(c >= 2)
            def _():
                pltpu.make_async_copy(
                    xout, out_hbm.at[pl.ds((row0 + (c - 2) * CHUNK) * D, CHUNK * D)],
                    osems[b]).wait()

            @pl.loop(0, CHUNK)
            def _row(ri):
                base = ri * D
                thresh = _select_threshold(xin, hist, cbuf, tref, base)

                @plsc.parallel_loop(0, VECS, unroll=8)
                def _write(v):
                    x = xin[pl.ds(base + v * L, L)]
                    a = lax.bitcast_convert_type(x, jnp.int32) & jnp.int32(0x7FFFFFFF)
                    mv = maskv[pl.ds(v * L, L)]
                    xout[pl.ds(base + v * L, L)] = jnp.where(a >= thresh, x * mv, 0.0)

            pltpu.async_copy(
                xout, out_hbm.at[pl.ds((row0 + c * CHUNK) * D, CHUNK * D)], osems[b])

            @pl.when(c + 2 < NCHUNKS)
            def _():
                pltpu.async_copy(_in_slice(c + 2), xin, isems[b])

    for b in range(2):
        pltpu.make_async_copy(
            xouts[b],
            out_hbm.at[pl.ds((row0 + (NCHUNKS - 2 + b) * CHUNK) * D, CHUNK * D)],
            osems[b]).wait()


_sc_topk = functools.partial(
    pl.kernel,
    out_type=jax.ShapeDtypeStruct((N_SC * D,), jnp.float32),
    mesh=plsc.VectorSubcoreMesh(core_axis_name="c", subcore_axis_name="s"),
    compiler_params=pltpu.CompilerParams(needs_layout_passes=False),
    scratch_types=[
        pltpu.VMEM((CHUNK * D,), jnp.float32),
        pltpu.VMEM((CHUNK * D,), jnp.float32),
        pltpu.VMEM((CHUNK * D,), jnp.float32),
        pltpu.VMEM((CHUNK * D,), jnp.float32),
        pltpu.VMEM((D,), jnp.float32),
        pltpu.VMEM((256,), jnp.int32),
        pltpu.VMEM((2 * L,), jnp.int32),
        pltpu.SMEM((8,), jnp.int32),
        pltpu.SemaphoreType.DMA,
        pltpu.SemaphoreType.DMA,
        pltpu.SemaphoreType.DMA,
        pltpu.SemaphoreType.DMA,
    ],
)(_tec_body)


BR = 256
N_TC = N_ROWS - N_SC


def _tc_block(x_ref, ml_ref, o_ref):
    x = x_ref[...]
    a = lax.bitcast_convert_type(x, jnp.int32) & jnp.int32(0x7FFFFFFF)
    lo = jnp.zeros((BR, 1), jnp.int32)
    hi = jnp.full((BR, 1), jnp.int32(0x7F800001))

    def body(_, lohi):
        lo, hi = lohi
        mid = lax.shift_right_logical(lo + hi, 1)
        cnt = jnp.sum((a >= mid).astype(jnp.int32), axis=1, keepdims=True)
        pred = cnt >= K
        return jnp.where(pred, mid, lo), jnp.where(pred, hi, mid)

    lo, hi = lax.fori_loop(0, 31, body, (lo, hi))
    mask = jax.nn.sigmoid(ml_ref[...])
    o_ref[...] = jnp.where(a >= lo, x * mask, 0.0)


_tc_topk = pl.pallas_call(
    _tc_block,
    grid=(N_TC // BR,),
    in_specs=[
        pl.BlockSpec((BR, D), lambda i: (N_SC // BR + i, 0)),
        pl.BlockSpec((1, D), lambda i: (0, 0)),
    ],
    out_specs=pl.BlockSpec((BR, D), lambda i: (i, 0)),
    out_shape=jax.ShapeDtypeStruct((N_TC, D), jnp.float32),
)


@jax.jit
def kernel(x, mask_logits):
    sc_out = _sc_topk(x.reshape(-1), mask_logits)
    tc_out = _tc_topk(x, mask_logits.reshape(1, D))
    return jnp.concatenate([sc_out.reshape(N_SC, D), tc_out], axis=0)
